# baseline (speedup 1.0000x reference)
import jax
import jax.numpy as jnp
from jax import lax
from jax.experimental import pallas as pl
from jax.experimental.pallas import tpu as pltpu
from jax.experimental.pallas import tpu_sc as plsc

VOCAB = 100000
EMBED = 128
BATCH = 16384

_INFO = plsc.get_sparse_core_info()
_NC, _NS, _L = _INFO.num_cores, _INFO.num_subcores, _INFO.num_lanes
_NW = _NC * _NS
_BPW = BATCH // _NW
_CHUNK = 128
_NCHUNK = _BPW // _CHUNK
_GROUPS = _CHUNK // 16


def _sc_body(word_hbm, ctx_hbm, wt_hbm, ct_hbm, dw_hbm, db_hbm, out_hbm,
             idx_w, idx_c, wrows, crows, tbuf, res, scal_v,
             sem_iw, sem_ic, sem_w, sem_c):
    wid = lax.axis_index("s") * _NC + lax.axis_index("c")
    base = wid * _BPW

    ciw0 = pltpu.async_copy(word_hbm.at[pl.ds(base, _CHUNK)],
                            idx_w.at[pl.ds(0, _CHUNK)], sem_iw)
    cic0 = pltpu.async_copy(ctx_hbm.at[pl.ds(base, _CHUNK)],
                            idx_c.at[pl.ds(0, _CHUNK)], sem_ic)
    ciw1 = pltpu.async_copy(word_hbm.at[pl.ds(base + _CHUNK, _BPW - _CHUNK)],
                            idx_w.at[pl.ds(_CHUNK, _BPW - _CHUNK)], sem_iw)
    cic1 = pltpu.async_copy(ctx_hbm.at[pl.ds(base + _CHUNK, _BPW - _CHUNK)],
                            idx_c.at[pl.ds(_CHUNK, _BPW - _CHUNK)], sem_ic)

    def issue(c, slot):
        off = c * _CHUNK
        pltpu.async_copy(wt_hbm.at[idx_w.at[pl.ds(off, _CHUNK)]],
                         wrows.at[slot], sem_w)
        pltpu.async_copy(ct_hbm.at[idx_c.at[pl.ds(off, _CHUNK)]],
                         crows.at[slot], sem_c)

    ciw0.wait()
    cic0.wait()
    issue(0, 0)
    ciw1.wait()
    cic1.wait()

    pltpu.sync_copy(dw_hbm, scal_v.at[pl.ds(0, 1)])
    pltpu.sync_copy(db_hbm, scal_v.at[pl.ds(8, 1)])
    sv = scal_v[pl.ds(0, 16)]
    wvec = jnp.broadcast_to(sv[0], (_L,))
    bvec = jnp.broadcast_to(sv[8], (_L,))
    iota16 = lax.iota(jnp.int32, _L)

    lomask = iota16 < 4

    def half_body(h, _):
        c = lax.shift_right_logical(h, 5)
        slot = lax.bitwise_and(c, 1)
        hin = lax.bitwise_and(h, 31)

        @pl.when(hin == 0)
        def _chunk_boundary():
            pltpu.make_async_copy(wt_hbm.at[idx_w.at[pl.ds(0, _CHUNK)]],
                                  wrows.at[0], sem_w).wait()
            pltpu.make_async_copy(ct_hbm.at[idx_c.at[pl.ds(0, _CHUNK)]],
                                  crows.at[0], sem_c).wait()

            @pl.when(c + 1 < _NCHUNK)
            def _issue_next():
                issue(c + 1, 1 - slot)

        row0 = hin * 4
        acc = jnp.zeros((_L,), jnp.float32)
        for r in range(4):
            row = row0 + r
            p = (wrows[slot, row, pl.ds(0, 16)] *
                 crows[slot, row, pl.ds(0, 16)])
            for k in range(1, EMBED // 16):
                p = p + (wrows[slot, row, pl.ds(k * 16, 16)] *
                         crows[slot, row, pl.ds(k * 16, 16)])
            acc = jnp.where(iota16 == r, lax.reduce_sum(p, (0,)), acc)
        z = acc * wvec + bvec
        out = 1.0 / (1.0 + jnp.exp(-z))
        plsc.store_scatter(res, [c * _CHUNK + row0 + iota16], out, mask=lomask)
        return 0

    lax.fori_loop(0, 4 * _NCHUNK * _GROUPS, half_body, 0)

    pltpu.sync_copy(res, out_hbm.at[pl.ds(base, _BPW)])


def kernel(word, context, word_table, ctx_table, dense_w, dense_b):
    word_i = word.reshape(-1).astype(jnp.int32)
    ctx_i = context.reshape(-1).astype(jnp.int32)
    dw = dense_w.reshape(-1).astype(jnp.float32)
    db = dense_b.reshape(-1).astype(jnp.float32)

    mesh = plsc.VectorSubcoreMesh(core_axis_name="c", subcore_axis_name="s")
    out = pl.kernel(
        _sc_body,
        out_type=jax.ShapeDtypeStruct((BATCH,), jnp.float32),
        mesh=mesh,
        compiler_params=pltpu.CompilerParams(
            needs_layout_passes=False,
            skip_device_barrier=True,
            disable_bounds_checks=True,
            disable_semaphore_checks=True,
        ),
        scratch_types=[
            pltpu.VMEM((_BPW,), jnp.int32),
            pltpu.VMEM((_BPW,), jnp.int32),
            pltpu.VMEM((2, _CHUNK, EMBED), jnp.float32),
            pltpu.VMEM((2, _CHUNK, EMBED), jnp.float32),
            pltpu.VMEM((256,), jnp.float32),
            pltpu.VMEM((_BPW,), jnp.float32),
            pltpu.VMEM((32,), jnp.float32),
            pltpu.SemaphoreType.DMA,
            pltpu.SemaphoreType.DMA,
            pltpu.SemaphoreType.DMA,
            pltpu.SemaphoreType.DMA,
        ],
    )(word_i, ctx_i, word_table, ctx_table, dw, db)
    return out.reshape(BATCH, 1)


# speedup vs baseline: 1.0356x; 1.0356x over previous
import jax
import jax.numpy as jnp
from jax import lax
from jax.experimental import pallas as pl
from jax.experimental.pallas import tpu as pltpu
from jax.experimental.pallas import tpu_sc as plsc

VOCAB = 100000
EMBED = 128
BATCH = 16384

_INFO = plsc.get_sparse_core_info()
_NC, _NS, _L = _INFO.num_cores, _INFO.num_subcores, _INFO.num_lanes
_NW = _NC * _NS
_BPW = BATCH // _NW
_CHUNK = 128
_NCHUNK = _BPW // _CHUNK
_GROUPS = _CHUNK // 16


def _sc_body(word_hbm, ctx_hbm, wt_hbm, ct_hbm, dw_hbm, db_hbm, out_hbm,
             idx_w, idx_c, wrows, crows, res, scal_v,
             sem_iw, sem_ic, sem_w, sem_c):
    wid = lax.axis_index("s") * _NC + lax.axis_index("c")
    base = wid * _BPW

    ciw0 = pltpu.async_copy(word_hbm.at[pl.ds(base, _CHUNK)],
                            idx_w.at[pl.ds(0, _CHUNK)], sem_iw)
    cic0 = pltpu.async_copy(ctx_hbm.at[pl.ds(base, _CHUNK)],
                            idx_c.at[pl.ds(0, _CHUNK)], sem_ic)
    ciw1 = pltpu.async_copy(word_hbm.at[pl.ds(base + _CHUNK, _BPW - _CHUNK)],
                            idx_w.at[pl.ds(_CHUNK, _BPW - _CHUNK)], sem_iw)
    cic1 = pltpu.async_copy(ctx_hbm.at[pl.ds(base + _CHUNK, _BPW - _CHUNK)],
                            idx_c.at[pl.ds(_CHUNK, _BPW - _CHUNK)], sem_ic)

    def issue(c, slot):
        off = c * _CHUNK
        pltpu.async_copy(wt_hbm.at[idx_w.at[pl.ds(off, _CHUNK)]],
                         wrows.at[slot], sem_w)
        pltpu.async_copy(ct_hbm.at[idx_c.at[pl.ds(off, _CHUNK)]],
                         crows.at[slot], sem_c)

    ciw0.wait()
    cic0.wait()
    issue(0, 0)
    ciw1.wait()
    cic1.wait()

    pltpu.sync_copy(dw_hbm, scal_v.at[pl.ds(0, 1)])
    pltpu.sync_copy(db_hbm, scal_v.at[pl.ds(8, 1)])
    sv = scal_v[pl.ds(0, 16)]
    wvec = jnp.broadcast_to(sv[0], (_L,))
    bvec = jnp.broadcast_to(sv[8], (_L,))
    iota16 = lax.iota(jnp.int32, _L)

    lomask = iota16 < 8

    def half_body(h, _):
        c = lax.shift_right_logical(h, 4)
        slot = lax.bitwise_and(c, 1)
        hin = lax.bitwise_and(h, 15)

        @pl.when(hin == 0)
        def _chunk_boundary():
            pltpu.make_async_copy(wt_hbm.at[idx_w.at[pl.ds(0, _CHUNK)]],
                                  wrows.at[0], sem_w).wait()
            pltpu.make_async_copy(ct_hbm.at[idx_c.at[pl.ds(0, _CHUNK)]],
                                  crows.at[0], sem_c).wait()

            @pl.when(c + 1 < _NCHUNK)
            def _issue_next():
                issue(c + 1, 1 - slot)

        row0 = hin * 8
        acc = jnp.zeros((_L,), jnp.float32)
        for r in range(8):
            row = row0 + r
            p = (wrows[slot, row, pl.ds(0, 16)] *
                 crows[slot, row, pl.ds(0, 16)])
            for k in range(1, EMBED // 16):
                p = p + (wrows[slot, row, pl.ds(k * 16, 16)] *
                         crows[slot, row, pl.ds(k * 16, 16)])
            acc = jnp.where(iota16 == r, lax.reduce_sum(p, (0,)), acc)
        z = acc * wvec + bvec
        out = 1.0 / (1.0 + jnp.exp(-z))
        plsc.store_scatter(res, [c * _CHUNK + row0 + iota16], out, mask=lomask)
        return 0

    lax.fori_loop(0, 2 * _NCHUNK * _GROUPS, half_body, 0)

    pltpu.sync_copy(res, out_hbm.at[pl.ds(base, _BPW)])


def kernel(word, context, word_table, ctx_table, dense_w, dense_b):
    word_i = word.reshape(-1).astype(jnp.int32)
    ctx_i = context.reshape(-1).astype(jnp.int32)
    dw = dense_w.reshape(-1).astype(jnp.float32)
    db = dense_b.reshape(-1).astype(jnp.float32)

    mesh = plsc.VectorSubcoreMesh(core_axis_name="c", subcore_axis_name="s")
    out = pl.kernel(
        _sc_body,
        out_type=jax.ShapeDtypeStruct((BATCH,), jnp.float32),
        mesh=mesh,
        compiler_params=pltpu.CompilerParams(
            needs_layout_passes=False,
            skip_device_barrier=True,
            disable_bounds_checks=True,
            disable_semaphore_checks=True,
        ),
        scratch_types=[
            pltpu.VMEM((_BPW,), jnp.int32),
            pltpu.VMEM((_BPW,), jnp.int32),
            pltpu.VMEM((2, _CHUNK, EMBED), jnp.float32),
            pltpu.VMEM((2, _CHUNK, EMBED), jnp.float32),
            pltpu.VMEM((_BPW,), jnp.float32),
            pltpu.VMEM((32,), jnp.float32),
            pltpu.SemaphoreType.DMA,
            pltpu.SemaphoreType.DMA,
            pltpu.SemaphoreType.DMA,
            pltpu.SemaphoreType.DMA,
        ],
    )(word_i, ctx_i, word_table, ctx_table, dw, db)
    return out.reshape(BATCH, 1)
